# baseline (speedup 1.0000x reference)
import functools

import jax
import jax.numpy as jnp
from jax import lax
from jax.experimental import pallas as pl
from jax.experimental.pallas import tpu as pltpu
from jax.experimental.pallas import tpu_sc as plsc

F = 128
NC = 2
NS = 16
NW = NC * NS


def _silu(x):
    return x * jax.nn.sigmoid(x)


def _pack_body(s_ref, v2_ref, tbl_ref):
    def u(x):
        return jax.lax.bitcast_convert_type(
            x.astype(jnp.bfloat16), jnp.uint16).astype(jnp.int32)

    sw = u(s_ref[...])
    v0 = u(v2_ref[:, 0 * F:1 * F])
    v1 = u(v2_ref[:, 1 * F:2 * F])
    v2w = u(v2_ref[:, 2 * F:3 * F])
    tbl_ref[:, :F] = sw | (v0 << 16)
    tbl_ref[:, F:] = v1 | (v2w << 16)


def _make_pack(N, BN):
    return pl.pallas_call(
        _pack_body,
        grid=(N // BN,),
        in_specs=[pl.BlockSpec((BN, F), lambda i: (i, 0)),
                  pl.BlockSpec((BN, 3 * F), lambda i: (i, 0))],
        out_specs=pl.BlockSpec((BN, 2 * F), lambda i: (i, 0)),
        out_shape=jax.ShapeDtypeStruct((N, 2 * F), jnp.int32))


def _make_gather(E, C, EBASE, ETOT):
    NCH = E // C
    NJ = -(-NCH // NW)
    NJ += NJ % 2
    mesh = plsc.VectorSubcoreMesh(core_axis_name="c", subcore_axis_name="s")

    W = 2 * F
    buf_set = [
        pltpu.VMEM((C,), jnp.int32),
        pltpu.VMEM((C,), jnp.int32),
        pltpu.VMEM((C, W), jnp.int32),
        pltpu.VMEM((C, W), jnp.int32),
        pltpu.SemaphoreType.DMA,
        pltpu.SemaphoreType.DMA,
        pltpu.SemaphoreType.DMA,
    ]

    @functools.partial(
        pl.kernel,
        mesh=mesh,
        out_type=[
            jax.ShapeDtypeStruct((E, W), jnp.int32),
            jax.ShapeDtypeStruct((E, W), jnp.int32),
        ],
        scratch_types=buf_set + buf_set,
    )
    def gather_kernel(tbl_hbm, ei_hbm, rowd_hbm, cold_hbm, *bufs):
        wid = lax.axis_index("s") * NC + lax.axis_index("c")
        sets = [tuple(bufs[7 * i:7 * i + 7]) for i in range(2)]

        def prefetch(j, idxr, idxc, rbuf, cbuf, isem, gsem, wsem):
            c = wid + NW * j

            @pl.when(c < NCH)
            def _():
                off = c * C
                pltpu.async_copy(ei_hbm.at[pl.ds(EBASE + off, C)], idxr, isem)
                pltpu.async_copy(ei_hbm.at[pl.ds(ETOT + EBASE + off, C)],
                                 idxc, isem)

        def launch(j, idxr, idxc, rbuf, cbuf, isem, gsem, wsem):
            cp = wid + NW * (j - 2)

            @pl.when((0 <= cp) & (cp < NCH))
            def _():
                offp = cp * C
                pltpu.make_async_copy(rbuf, rowd_hbm.at[pl.ds(offp, C)],
                                      wsem).wait()
                pltpu.make_async_copy(cbuf, cold_hbm.at[pl.ds(offp, C)],
                                      wsem).wait()

            c = wid + NW * j

            @pl.when(c < NCH)
            def _():
                off = c * C
                pltpu.make_async_copy(ei_hbm.at[pl.ds(EBASE + off, C)],
                                      idxr, isem).wait()
                pltpu.make_async_copy(ei_hbm.at[pl.ds(EBASE + off, C)],
                                      idxc, isem).wait()
                pltpu.async_copy(tbl_hbm.at[idxr], rbuf, gsem)
                pltpu.async_copy(tbl_hbm.at[idxc], cbuf, gsem)

        def finish(j, idxr, idxc, rbuf, cbuf, isem, gsem, wsem):
            c = wid + NW * j

            @pl.when(c < NCH)
            def _():
                off = c * C
                pltpu.make_async_copy(tbl_hbm.at[idxr], rbuf, gsem).wait()
                pltpu.make_async_copy(tbl_hbm.at[idxc], cbuf, gsem).wait()
                pltpu.async_copy(rbuf, rowd_hbm.at[pl.ds(off, C)], wsem)
                pltpu.async_copy(cbuf, cold_hbm.at[pl.ds(off, C)], wsem)

        A, B = sets
        prefetch(0, *A)
        prefetch(1, *B)
        launch(0, *A)
        launch(1, *B)

        @pl.loop(0, NJ, step=2)
        def _(jj):
            finish(jj, *A)
            prefetch(jj + 2, *A)
            finish(jj + 1, *B)
            prefetch(jj + 3, *B)
            launch(jj + 2, *A)
            launch(jj + 3, *B)

    return gather_kernel


def _lo(x):
    return jax.lax.bitcast_convert_type(x << 16, jnp.float32)


def _hi(x):
    return jax.lax.bitcast_convert_type(x & jnp.int32(-65536), jnp.float32)


def _edge_body(rowd_ref, cold_ref, rbf_ref,
               msW1s_ref, msW1r_ref, msb1_ref, msW2_ref, msb2_ref,
               mvW1s_ref, mvW1r_ref, mvb1_ref, mvW2_ref, mvb2_ref,
               ds_ref, dv0_ref, dv1_ref, dv2_ref):
    rowa = rowd_ref[:, :F]
    rowb = rowd_ref[:, F:]
    cola = cold_ref[:, :F]
    colb = cold_ref[:, F:]

    x = _lo(rowa).astype(jnp.bfloat16)
    r = rbf_ref[...]
    h = jnp.dot(x, msW1s_ref[...], preferred_element_type=jnp.float32)
    h += jnp.dot(r, msW1r_ref[...], preferred_element_type=jnp.float32)
    h = _silu(h + msb1_ref[...]).astype(jnp.bfloat16)
    ds_ref[...] = (jnp.dot(h, msW2_ref[...], preferred_element_type=jnp.float32)
                   + msb2_ref[...])

    g = jnp.dot(x, mvW1s_ref[...], preferred_element_type=jnp.float32)
    g += jnp.dot(r, mvW1r_ref[...], preferred_element_type=jnp.float32)
    g = _silu(g + mvb1_ref[...]).astype(jnp.bfloat16)
    dvb = (jnp.dot(g, mvW2_ref[...], preferred_element_type=jnp.float32)
           + mvb2_ref[...])

    vd0 = _hi(rowa) - _hi(cola)
    vd1 = _lo(rowb) - _lo(colb)
    vd2 = _hi(rowb) - _hi(colb)
    nrm = jnp.sqrt(vd0 * vd0 + vd1 * vd1 + vd2 * vd2)
    scale = dvb / (nrm + 1e-8)
    dv0_ref[...] = scale * vd0
    dv1_ref[...] = scale * vd1
    dv2_ref[...] = scale * vd2


def _make_edge_call(E, BE):
    grid = (E // BE,)

    def full(shape):
        return pl.BlockSpec(shape, lambda i: (0,) * len(shape))

    in_specs = [
        pl.BlockSpec((BE, 2 * F), lambda i: (i, 0)),
        pl.BlockSpec((BE, 2 * F), lambda i: (i, 0)),
        pl.BlockSpec((BE, 16), lambda i: (i, 0)),
        full((F, F)), full((16, F)), full((1, F)), full((F, F)), full((1, F)),
        full((F, F)), full((16, F)), full((1, F)), full((F, F)), full((1, F)),
    ]
    out_specs = [pl.BlockSpec((BE, F), lambda i: (i, 0))] * 4
    out_shape = [jax.ShapeDtypeStruct((E, F), jnp.float32)] * 4
    return pl.pallas_call(
        _edge_body, grid=grid, in_specs=in_specs, out_specs=out_specs,
        out_shape=out_shape)


def _make_scatter(EH, N, C):
    NCH = EH // C
    NJ = -(-NCH // NW)
    NJ += NJ % 2
    RSUB = (N // NS) // 8 * 8
    RREM = N - NS * RSUB
    mesh = plsc.VectorSubcoreMesh(core_axis_name="c", subcore_axis_name="s")

    buf_set = [
        pltpu.VMEM((C,), jnp.int32),
        pltpu.VMEM((C, F), jnp.float32),
        pltpu.SemaphoreType.DMA,
    ]

    @functools.partial(
        pl.kernel,
        mesh=mesh,
        out_type=[jax.ShapeDtypeStruct((NC, N, F), jnp.float32)] * 4,
        scratch_types=buf_set + buf_set + [
            pltpu.VMEM_SHARED((N, F), jnp.float32),
        ],
    )
    def scatter_kernel(d0a, d0b, d1a, d1b, d2a, d2b, d3a, d3b,
                       ei_hbm, zero_hbm,
                       o0, o1, o2, o3,
                       idx0, dbuf0, sem0, idx1, dbuf1, sem1, acc):
        cid = lax.axis_index("c")
        sid = lax.axis_index("s")
        wid = sid * NC + cid
        rbase = sid * RSUB

        def accumulate(data_hbm, ebase):
            def start(j, idx, dbuf, sem):
                c = wid + NW * j

                @pl.when(c < NCH)
                def _():
                    off = 2 * EH + ebase + c * C
                    pltpu.async_copy(ei_hbm.at[pl.ds(off, C)], idx, sem)
                    pltpu.async_copy(data_hbm.at[pl.ds(c * C, C)], dbuf, sem)

            def finish(j, idx, dbuf, sem):
                c = wid + NW * j

                @pl.when(c < NCH)
                def _():
                    pltpu.make_async_copy(ei_hbm.at[pl.ds(0, C)], idx,
                                          sem).wait()
                    pltpu.make_async_copy(data_hbm.at[pl.ds(0, C)], dbuf,
                                          sem).wait()
                    pltpu.sync_copy(dbuf, acc.at[idx], add=True)

            A = (idx0, dbuf0, sem0)
            B = (idx1, dbuf1, sem1)
            start(0, *A)

            @pl.loop(0, NJ, step=2)
            def _(jj):
                start(jj + 1, *B)
                finish(jj, *A)
                start(jj + 2, *A)
                finish(jj + 1, *B)

        for (da, db), out_hbm in (((d0a, d0b), o0), ((d1a, d1b), o1),
                                  ((d2a, d2b), o2), ((d3a, d3b), o3)):
            pltpu.sync_copy(zero_hbm.at[pl.ds(0, RSUB)],
                            acc.at[pl.ds(rbase, RSUB)])

            @pl.when(sid == NS - 1)
            def _():
                pltpu.sync_copy(zero_hbm.at[pl.ds(0, RREM)],
                                acc.at[pl.ds(NS * RSUB, RREM)])

            plsc.subcore_barrier()
            accumulate(da, 0)
            accumulate(db, EH)
            plsc.subcore_barrier()

            pltpu.sync_copy(acc.at[pl.ds(rbase, RSUB)],
                            out_hbm.at[cid].at[pl.ds(rbase, RSUB)])

            @pl.when(sid == NS - 1)
            def _():
                pltpu.sync_copy(acc.at[pl.ds(NS * RSUB, RREM)],
                                out_hbm.at[cid].at[pl.ds(NS * RSUB, RREM)])

            plsc.subcore_barrier()

    return scatter_kernel


def _node_body(s_ref, v2_ref, a0_ref, a1_ref, a2_ref, a3_ref,
               usW1a_ref, usW1b_ref, usb1_ref, usW2_ref, usb2_ref,
               uvW1a_ref, uvW1b_ref, uvb1_ref, uvW2_ref, uvb2_ref,
               sout_ref, vout_ref):
    sv = s_ref[...]
    dsa = a0_ref[0] + a0_ref[1]
    h = jnp.dot(sv, usW1a_ref[...], preferred_element_type=jnp.float32)
    h += jnp.dot(dsa, usW1b_ref[...], preferred_element_type=jnp.float32)
    h = _silu(h + usb1_ref[...])
    sout_ref[...] = sv + (jnp.dot(h, usW2_ref[...],
                                  preferred_element_type=jnp.float32)
                          + usb2_ref[...])

    for k, ak_ref in enumerate((a1_ref, a2_ref, a3_ref)):
        vk = v2_ref[:, k * F:(k + 1) * F]
        dvk = ak_ref[0] + ak_ref[1]
        g = jnp.dot(vk, uvW1a_ref[...], preferred_element_type=jnp.float32)
        g += jnp.dot(dvk, uvW1b_ref[...], preferred_element_type=jnp.float32)
        g = _silu(g + uvb1_ref[...])
        vout_ref[:, k * F:(k + 1) * F] = vk + (
            jnp.dot(g, uvW2_ref[...], preferred_element_type=jnp.float32)
            + uvb2_ref[...])


def _make_node_call(N, BN):
    grid = (N // BN,)

    def full(shape):
        return pl.BlockSpec(shape, lambda i: (0,) * len(shape))

    in_specs = [
        pl.BlockSpec((BN, F), lambda i: (i, 0)),
        pl.BlockSpec((BN, 3 * F), lambda i: (i, 0)),
        pl.BlockSpec((NC, BN, F), lambda i: (0, i, 0)),
        pl.BlockSpec((NC, BN, F), lambda i: (0, i, 0)),
        pl.BlockSpec((NC, BN, F), lambda i: (0, i, 0)),
        pl.BlockSpec((NC, BN, F), lambda i: (0, i, 0)),
        full((F, F)), full((F, F)), full((1, F)), full((F, F)), full((1, F)),
        full((F, F)), full((F, F)), full((1, F)), full((F, F)), full((1, F)),
    ]
    out_specs = [
        pl.BlockSpec((BN, F), lambda i: (i, 0)),
        pl.BlockSpec((BN, 3 * F), lambda i: (i, 0)),
    ]
    out_shape = [
        jax.ShapeDtypeStruct((N, F), jnp.float32),
        jax.ShapeDtypeStruct((N, 3 * F), jnp.float32),
    ]
    return pl.pallas_call(
        _node_body, grid=grid, in_specs=in_specs, out_specs=out_specs,
        out_shape=out_shape)


def kernel(s, v, edge_index, rbf,
           msW1, msb1, msW2, msb2,
           mvW1, mvb1, mvW2, mvb2,
           usW1, usb1, usW2, usb2,
           uvW1, uvb1, uvW2, uvb2):
    N = s.shape[0]
    E = edge_index.shape[1]
    v2 = v.reshape(N, 3 * F)

    bf = jnp.bfloat16
    tbl = _make_pack(N, 2000)(s, v2)
    ei_flat = edge_index.reshape(2 * E)

    def t(W):
        return W.T

    EH = E // 2
    edge = _make_edge_call(EH, 1600)
    rbf_b = rbf.astype(bf)
    edge_w = (
        t(msW1)[:F].astype(bf), t(msW1)[F:].astype(bf), msb1.reshape(1, F),
        t(msW2).astype(bf), msb2.reshape(1, F),
        t(mvW1)[:F].astype(bf), t(mvW1)[F:].astype(bf), mvb1.reshape(1, F),
        t(mvW2).astype(bf), mvb2.reshape(1, F),
    )
    halves = []
    for h in range(2):
        rowdat, coldat = _make_gather(EH, 80, h * EH, E)(tbl, ei_flat)
        halves.append(
            edge(rowdat, coldat, lax.slice_in_dim(rbf_b, h * EH, (h + 1) * EH),
                 *edge_w))

    zero = jnp.zeros(((N // NS) // 8 * 8, F), jnp.float32)
    a0, a1, a2, a3 = _make_scatter(EH, N, 128)(
        halves[0][0], halves[1][0], halves[0][1], halves[1][1],
        halves[0][2], halves[1][2], halves[0][3], halves[1][3],
        ei_flat, zero)

    s_out, v2_out = _make_node_call(N, 2000)(
        s, v2, a0, a1, a2, a3,
        t(usW1)[:F], t(usW1)[F:], usb1.reshape(1, F), t(usW2), usb2.reshape(1, F),
        t(uvW1)[:F], t(uvW1)[F:], uvb1.reshape(1, F), t(uvW2), uvb2.reshape(1, F),
    )
    return s_out, v2_out.reshape(N, 3, F)


# speedup vs baseline: 1.0093x; 1.0093x over previous
import functools

import jax
import jax.numpy as jnp
from jax import lax
from jax.experimental import pallas as pl
from jax.experimental.pallas import tpu as pltpu
from jax.experimental.pallas import tpu_sc as plsc

F = 128
NC = 2
NS = 16
NW = NC * NS


def _silu(x):
    return x * jax.nn.sigmoid(x)


def _make_gather(E, C, EBASE, ETOT):
    NCH = E // C
    NJ = -(-NCH // NW)
    NJ += NJ % 2
    mesh = plsc.VectorSubcoreMesh(core_axis_name="c", subcore_axis_name="s")

    W = 2 * F
    buf_set = [
        pltpu.VMEM((C,), jnp.int32),
        pltpu.VMEM((C,), jnp.int32),
        pltpu.VMEM((C, W), jnp.int32),
        pltpu.VMEM((C, W), jnp.int32),
        pltpu.SemaphoreType.DMA,
        pltpu.SemaphoreType.DMA,
        pltpu.SemaphoreType.DMA,
    ]

    @functools.partial(
        pl.kernel,
        mesh=mesh,
        out_type=[
            jax.ShapeDtypeStruct((E, W), jnp.int32),
            jax.ShapeDtypeStruct((E, W), jnp.int32),
        ],
        scratch_types=buf_set + buf_set,
    )
    def gather_kernel(tbl_hbm, ei_hbm, rowd_hbm, cold_hbm, *bufs):
        wid = lax.axis_index("s") * NC + lax.axis_index("c")
        sets = [tuple(bufs[7 * i:7 * i + 7]) for i in range(2)]

        def prefetch(j, idxr, idxc, rbuf, cbuf, isem, gsem, wsem):
            c = wid + NW * j

            @pl.when(c < NCH)
            def _():
                off = c * C
                pltpu.async_copy(ei_hbm.at[pl.ds(EBASE + off, C)], idxr, isem)
                pltpu.async_copy(ei_hbm.at[pl.ds(ETOT + EBASE + off, C)],
                                 idxc, isem)

        def launch(j, idxr, idxc, rbuf, cbuf, isem, gsem, wsem):
            cp = wid + NW * (j - 2)

            @pl.when((0 <= cp) & (cp < NCH))
            def _():
                offp = cp * C
                pltpu.make_async_copy(rbuf, rowd_hbm.at[pl.ds(offp, C)],
                                      wsem).wait()
                pltpu.make_async_copy(cbuf, cold_hbm.at[pl.ds(offp, C)],
                                      wsem).wait()

            c = wid + NW * j

            @pl.when(c < NCH)
            def _():
                off = c * C
                pltpu.make_async_copy(ei_hbm.at[pl.ds(EBASE + off, C)],
                                      idxr, isem).wait()
                pltpu.make_async_copy(ei_hbm.at[pl.ds(EBASE + off, C)],
                                      idxc, isem).wait()
                pltpu.async_copy(tbl_hbm.at[idxr], rbuf, gsem)
                pltpu.async_copy(tbl_hbm.at[idxc], cbuf, gsem)

        def finish(j, idxr, idxc, rbuf, cbuf, isem, gsem, wsem):
            c = wid + NW * j

            @pl.when(c < NCH)
            def _():
                off = c * C
                pltpu.make_async_copy(tbl_hbm.at[idxr], rbuf, gsem).wait()
                pltpu.make_async_copy(tbl_hbm.at[idxc], cbuf, gsem).wait()
                pltpu.async_copy(rbuf, rowd_hbm.at[pl.ds(off, C)], wsem)
                pltpu.async_copy(cbuf, cold_hbm.at[pl.ds(off, C)], wsem)

        A, B = sets
        prefetch(0, *A)
        prefetch(1, *B)
        launch(0, *A)
        launch(1, *B)

        @pl.loop(0, NJ, step=2)
        def _(jj):
            finish(jj, *A)
            prefetch(jj + 2, *A)
            finish(jj + 1, *B)
            prefetch(jj + 3, *B)
            launch(jj + 2, *A)
            launch(jj + 3, *B)

    return gather_kernel


def _lo(x):
    return jax.lax.bitcast_convert_type(x << 16, jnp.float32)


def _hi(x):
    return jax.lax.bitcast_convert_type(x & jnp.int32(-65536), jnp.float32)


def _edge_body(rowd_ref, cold_ref, rbf_ref,
               msW1s_ref, msW1r_ref, msb1_ref, msW2_ref, msb2_ref,
               mvW1s_ref, mvW1r_ref, mvb1_ref, mvW2_ref, mvb2_ref,
               ds_ref, dv0_ref, dv1_ref, dv2_ref):
    rowa = rowd_ref[:, :F]
    rowb = rowd_ref[:, F:]
    cola = cold_ref[:, :F]
    colb = cold_ref[:, F:]

    x = _lo(rowa).astype(jnp.bfloat16)
    r = rbf_ref[...]
    h = jnp.dot(x, msW1s_ref[...], preferred_element_type=jnp.float32)
    h += jnp.dot(r, msW1r_ref[...], preferred_element_type=jnp.float32)
    h = _silu(h + msb1_ref[...]).astype(jnp.bfloat16)
    ds_ref[...] = (jnp.dot(h, msW2_ref[...], preferred_element_type=jnp.float32)
                   + msb2_ref[...])

    g = jnp.dot(x, mvW1s_ref[...], preferred_element_type=jnp.float32)
    g += jnp.dot(r, mvW1r_ref[...], preferred_element_type=jnp.float32)
    g = _silu(g + mvb1_ref[...]).astype(jnp.bfloat16)
    dvb = (jnp.dot(g, mvW2_ref[...], preferred_element_type=jnp.float32)
           + mvb2_ref[...])

    vd0 = _hi(rowa) - _hi(cola)
    vd1 = _lo(rowb) - _lo(colb)
    vd2 = _hi(rowb) - _hi(colb)
    nrm = jnp.sqrt(vd0 * vd0 + vd1 * vd1 + vd2 * vd2)
    scale = dvb / (nrm + 1e-8)
    dv0_ref[...] = scale * vd0
    dv1_ref[...] = scale * vd1
    dv2_ref[...] = scale * vd2


def _make_edge_call(E, BE):
    grid = (E // BE,)

    def full(shape):
        return pl.BlockSpec(shape, lambda i: (0,) * len(shape))

    in_specs = [
        pl.BlockSpec((BE, 2 * F), lambda i: (i, 0)),
        pl.BlockSpec((BE, 2 * F), lambda i: (i, 0)),
        pl.BlockSpec((BE, 16), lambda i: (i, 0)),
        full((F, F)), full((16, F)), full((1, F)), full((F, F)), full((1, F)),
        full((F, F)), full((16, F)), full((1, F)), full((F, F)), full((1, F)),
    ]
    out_specs = [pl.BlockSpec((BE, F), lambda i: (i, 0))] * 4
    out_shape = [jax.ShapeDtypeStruct((E, F), jnp.float32)] * 4
    return pl.pallas_call(
        _edge_body, grid=grid, in_specs=in_specs, out_specs=out_specs,
        out_shape=out_shape)


def _make_scatter(EH, N, C):
    NCH = EH // C
    NJ = -(-NCH // NW)
    NJ += NJ % 2
    RSUB = (N // NS) // 8 * 8
    RREM = N - NS * RSUB
    mesh = plsc.VectorSubcoreMesh(core_axis_name="c", subcore_axis_name="s")

    buf_set = [
        pltpu.VMEM((C,), jnp.int32),
        pltpu.VMEM((C, F), jnp.float32),
        pltpu.SemaphoreType.DMA,
    ]

    @functools.partial(
        pl.kernel,
        mesh=mesh,
        out_type=[jax.ShapeDtypeStruct((NC, N, F), jnp.float32)] * 4,
        scratch_types=buf_set + buf_set + [
            pltpu.VMEM_SHARED((N, F), jnp.float32),
        ],
    )
    def scatter_kernel(d0a, d0b, d1a, d1b, d2a, d2b, d3a, d3b,
                       ei_hbm, zero_hbm,
                       o0, o1, o2, o3,
                       idx0, dbuf0, sem0, idx1, dbuf1, sem1, acc):
        cid = lax.axis_index("c")
        sid = lax.axis_index("s")
        wid = sid * NC + cid
        rbase = sid * RSUB

        def accumulate(data_hbm, ebase):
            def start(j, idx, dbuf, sem):
                c = wid + NW * j

                @pl.when(c < NCH)
                def _():
                    off = 2 * EH + ebase + c * C
                    pltpu.async_copy(ei_hbm.at[pl.ds(off, C)], idx, sem)
                    pltpu.async_copy(data_hbm.at[pl.ds(c * C, C)], dbuf, sem)

            def finish(j, idx, dbuf, sem):
                c = wid + NW * j

                @pl.when(c < NCH)
                def _():
                    pltpu.make_async_copy(ei_hbm.at[pl.ds(0, C)], idx,
                                          sem).wait()
                    pltpu.make_async_copy(data_hbm.at[pl.ds(0, C)], dbuf,
                                          sem).wait()
                    pltpu.sync_copy(dbuf, acc.at[idx], add=True)

            A = (idx0, dbuf0, sem0)
            B = (idx1, dbuf1, sem1)
            start(0, *A)

            @pl.loop(0, NJ, step=2)
            def _(jj):
                start(jj + 1, *B)
                finish(jj, *A)
                start(jj + 2, *A)
                finish(jj + 1, *B)

        for (da, db), out_hbm in (((d0a, d0b), o0), ((d1a, d1b), o1),
                                  ((d2a, d2b), o2), ((d3a, d3b), o3)):
            pltpu.sync_copy(zero_hbm.at[pl.ds(0, RSUB)],
                            acc.at[pl.ds(rbase, RSUB)])

            @pl.when(sid == NS - 1)
            def _():
                pltpu.sync_copy(zero_hbm.at[pl.ds(0, RREM)],
                                acc.at[pl.ds(NS * RSUB, RREM)])

            plsc.subcore_barrier()
            accumulate(da, 0)
            accumulate(db, EH)
            plsc.subcore_barrier()

            pltpu.sync_copy(acc.at[pl.ds(rbase, RSUB)],
                            out_hbm.at[cid].at[pl.ds(rbase, RSUB)])

            @pl.when(sid == NS - 1)
            def _():
                pltpu.sync_copy(acc.at[pl.ds(NS * RSUB, RREM)],
                                out_hbm.at[cid].at[pl.ds(NS * RSUB, RREM)])

            plsc.subcore_barrier()

    return scatter_kernel


def _node_body(s_ref, v2_ref, a0_ref, a1_ref, a2_ref, a3_ref,
               usW1a_ref, usW1b_ref, usb1_ref, usW2_ref, usb2_ref,
               uvW1a_ref, uvW1b_ref, uvb1_ref, uvW2_ref, uvb2_ref,
               sout_ref, vout_ref):
    sv = s_ref[...]
    dsa = a0_ref[0] + a0_ref[1]
    h = jnp.dot(sv, usW1a_ref[...], preferred_element_type=jnp.float32)
    h += jnp.dot(dsa, usW1b_ref[...], preferred_element_type=jnp.float32)
    h = _silu(h + usb1_ref[...])
    sout_ref[...] = sv + (jnp.dot(h, usW2_ref[...],
                                  preferred_element_type=jnp.float32)
                          + usb2_ref[...])

    for k, ak_ref in enumerate((a1_ref, a2_ref, a3_ref)):
        vk = v2_ref[:, k * F:(k + 1) * F]
        dvk = ak_ref[0] + ak_ref[1]
        g = jnp.dot(vk, uvW1a_ref[...], preferred_element_type=jnp.float32)
        g += jnp.dot(dvk, uvW1b_ref[...], preferred_element_type=jnp.float32)
        g = _silu(g + uvb1_ref[...])
        vout_ref[:, k * F:(k + 1) * F] = vk + (
            jnp.dot(g, uvW2_ref[...], preferred_element_type=jnp.float32)
            + uvb2_ref[...])


def _make_node_call(N, BN):
    grid = (N // BN,)

    def full(shape):
        return pl.BlockSpec(shape, lambda i: (0,) * len(shape))

    in_specs = [
        pl.BlockSpec((BN, F), lambda i: (i, 0)),
        pl.BlockSpec((BN, 3 * F), lambda i: (i, 0)),
        pl.BlockSpec((NC, BN, F), lambda i: (0, i, 0)),
        pl.BlockSpec((NC, BN, F), lambda i: (0, i, 0)),
        pl.BlockSpec((NC, BN, F), lambda i: (0, i, 0)),
        pl.BlockSpec((NC, BN, F), lambda i: (0, i, 0)),
        full((F, F)), full((F, F)), full((1, F)), full((F, F)), full((1, F)),
        full((F, F)), full((F, F)), full((1, F)), full((F, F)), full((1, F)),
    ]
    out_specs = [
        pl.BlockSpec((BN, F), lambda i: (i, 0)),
        pl.BlockSpec((BN, 3 * F), lambda i: (i, 0)),
    ]
    out_shape = [
        jax.ShapeDtypeStruct((N, F), jnp.float32),
        jax.ShapeDtypeStruct((N, 3 * F), jnp.float32),
    ]
    return pl.pallas_call(
        _node_body, grid=grid, in_specs=in_specs, out_specs=out_specs,
        out_shape=out_shape)


def kernel(s, v, edge_index, rbf,
           msW1, msb1, msW2, msb2,
           mvW1, mvb1, mvW2, mvb2,
           usW1, usb1, usW2, usb2,
           uvW1, uvb1, uvW2, uvb2):
    N = s.shape[0]
    E = edge_index.shape[1]
    v2 = v.reshape(N, 3 * F)

    bf = jnp.bfloat16
    sb = s.astype(bf)
    vb = v.astype(bf)
    tbl = jnp.concatenate([
        jax.lax.bitcast_convert_type(
            jnp.stack([sb, vb[:, 0]], axis=-1), jnp.int32),
        jax.lax.bitcast_convert_type(
            jnp.stack([vb[:, 1], vb[:, 2]], axis=-1), jnp.int32),
    ], axis=1)
    ei_flat = edge_index.reshape(2 * E)

    def t(W):
        return W.T

    EH = E // 2
    edge = _make_edge_call(EH, 1600)
    rbf_b = rbf.astype(bf)
    edge_w = (
        t(msW1)[:F].astype(bf), t(msW1)[F:].astype(bf), msb1.reshape(1, F),
        t(msW2).astype(bf), msb2.reshape(1, F),
        t(mvW1)[:F].astype(bf), t(mvW1)[F:].astype(bf), mvb1.reshape(1, F),
        t(mvW2).astype(bf), mvb2.reshape(1, F),
    )
    halves = []
    for h in range(2):
        rowdat, coldat = _make_gather(EH, 80, h * EH, E)(tbl, ei_flat)
        halves.append(
            edge(rowdat, coldat, lax.slice_in_dim(rbf_b, h * EH, (h + 1) * EH),
                 *edge_w))

    zero = jnp.zeros(((N // NS) // 8 * 8, F), jnp.float32)
    a0, a1, a2, a3 = _make_scatter(EH, N, 128)(
        halves[0][0], halves[1][0], halves[0][1], halves[1][1],
        halves[0][2], halves[1][2], halves[0][3], halves[1][3],
        ei_flat, zero)

    s_out, v2_out = _make_node_call(N, 2000)(
        s, v2, a0, a1, a2, a3,
        t(usW1)[:F], t(usW1)[F:], usb1.reshape(1, F), t(usW2), usb2.reshape(1, F),
        t(uvW1)[:F], t(uvW1)[F:], uvb1.reshape(1, F), t(uvW2), uvb2.reshape(1, F),
    )
    return s_out, v2_out.reshape(N, 3, F)
